# baseline (speedup 1.0000x reference)
import math

import jax
import jax.numpy as jnp
from jax.experimental import pallas as pl
from jax.experimental.pallas import tpu as pltpu

B = 16
Z = 4096
D = 768
K = 32
ZC = 1024
NZ = Z // ZC
SCALE = 1.0 / math.sqrt(D)


def _logits_kernel(txt_ref, zone_ref, wt_ref, bt_ref, wz_ref, bz_ref, out_ref):
    b = pl.program_id(0)
    txt_b = txt_ref[pl.ds(b, 1), :]
    t = jnp.tanh(
        jax.lax.dot_general(txt_b, wt_ref[...], (((1,), (1,)), ((), ())),
                            preferred_element_type=jnp.float32)
        + bt_ref[...]
    )
    z = jax.lax.dot_general(zone_ref[0], wz_ref[...], (((1,), (1,)), ((), ())),
                            preferred_element_type=jnp.float32)
    z = jnp.tanh(z + bz_ref[...])
    logits = jnp.sum(z * t, axis=1) * SCALE
    out_ref[...] = logits[None, None, :]


def _softmax_kernel(logits_ref, mask_ref, out_ref):
    x = jnp.where(mask_ref[...] != 0, -jnp.inf, logits_ref[...])
    m = jnp.max(x, axis=1, keepdims=True)
    e = jnp.exp(x - m)
    out_ref[...] = e / jnp.sum(e, axis=1, keepdims=True)


def kernel(txt_embeds, zone_embeds, W_txt, b_txt, W_zone, b_zone, mask):
    logits = pl.pallas_call(
        _logits_kernel,
        grid=(B, NZ),
        in_specs=[
            pl.BlockSpec((B, D), lambda b, j: (0, 0)),
            pl.BlockSpec((1, ZC, D), lambda b, j: (b, j, 0)),
            pl.BlockSpec((K, D), lambda b, j: (0, 0)),
            pl.BlockSpec((1, K), lambda b, j: (0, 0)),
            pl.BlockSpec((K, D), lambda b, j: (0, 0)),
            pl.BlockSpec((1, K), lambda b, j: (0, 0)),
        ],
        out_specs=pl.BlockSpec((1, 1, ZC), lambda b, j: (b * NZ + j, 0, 0)),
        out_shape=jax.ShapeDtypeStruct((B * NZ, 1, ZC), jnp.float32),
        compiler_params=pltpu.CompilerParams(
            dimension_semantics=("parallel", "parallel")),
    )(txt_embeds, zone_embeds, W_txt, b_txt.reshape(1, K),
      W_zone, b_zone.reshape(1, K))
    return pl.pallas_call(
        _softmax_kernel,
        out_shape=jax.ShapeDtypeStruct((B, Z), jnp.float32),
    )(logits.reshape(B, Z), mask.astype(jnp.int32))


# speedup vs baseline: 1.2054x; 1.2054x over previous
import math

import jax
import jax.numpy as jnp
from jax.experimental import pallas as pl
from jax.experimental.pallas import tpu as pltpu

B = 16
Z = 4096
D = 768
K = 32
ZC = 1024
NZ = Z // ZC
SCALE = 1.0 / math.sqrt(D)


def _logits_kernel(txt_ref, zone_ref, wt_ref, bt_ref, wz_ref, bz_ref, out_ref):
    b = pl.program_id(0)
    txt_b = txt_ref[pl.ds(b, 1), :]
    t = jnp.tanh(
        jax.lax.dot_general(txt_b, wt_ref[...], (((1,), (1,)), ((), ())),
                            preferred_element_type=jnp.float32)
        + bt_ref[...]
    )
    z = jax.lax.dot_general(zone_ref[0], wz_ref[...], (((1,), (1,)), ((), ())),
                            preferred_element_type=jnp.float32)
    z = jnp.tanh(z + bz_ref[...])
    logits = jax.lax.dot_general(t * SCALE, z, (((1,), (1,)), ((), ())),
                                 preferred_element_type=jnp.float32)
    out_ref[...] = logits[None]


def _softmax_kernel(logits_ref, mask_ref, out_ref):
    x = jnp.where(mask_ref[...] != 0, -jnp.inf, logits_ref[...])
    m = jnp.max(x, axis=1, keepdims=True)
    e = jnp.exp(x - m)
    out_ref[...] = e / jnp.sum(e, axis=1, keepdims=True)


def kernel(txt_embeds, zone_embeds, W_txt, b_txt, W_zone, b_zone, mask):
    logits = pl.pallas_call(
        _logits_kernel,
        grid=(B, NZ),
        in_specs=[
            pl.BlockSpec((B, D), lambda b, j: (0, 0)),
            pl.BlockSpec((1, ZC, D), lambda b, j: (b, j, 0)),
            pl.BlockSpec((K, D), lambda b, j: (0, 0)),
            pl.BlockSpec((1, K), lambda b, j: (0, 0)),
            pl.BlockSpec((K, D), lambda b, j: (0, 0)),
            pl.BlockSpec((1, K), lambda b, j: (0, 0)),
        ],
        out_specs=pl.BlockSpec((1, 1, ZC), lambda b, j: (b * NZ + j, 0, 0)),
        out_shape=jax.ShapeDtypeStruct((B * NZ, 1, ZC), jnp.float32),
        compiler_params=pltpu.CompilerParams(
            dimension_semantics=("parallel", "parallel")),
    )(txt_embeds, zone_embeds, W_txt, b_txt.reshape(1, K),
      W_zone, b_zone.reshape(1, K))
    return pl.pallas_call(
        _softmax_kernel,
        out_shape=jax.ShapeDtypeStruct((B, Z), jnp.float32),
    )(logits.reshape(B, Z), mask.astype(jnp.int32))


# speedup vs baseline: 1.7196x; 1.4266x over previous
import math

import jax
import jax.numpy as jnp
from jax.experimental import pallas as pl
from jax.experimental.pallas import tpu as pltpu

B = 16
Z = 4096
D = 768
K = 32
SCALE = 1.0 / math.sqrt(D)


def _fused_kernel(txt_ref, zone_ref, wt_ref, bt_ref, wz_ref, bz_ref,
                  mask_ref, out_ref):
    b = pl.program_id(0)
    txt_b = txt_ref[pl.ds(b, 1), :]
    t = jnp.tanh(
        jax.lax.dot_general(txt_b, wt_ref[...], (((1,), (1,)), ((), ())),
                            preferred_element_type=jnp.float32)
        + bt_ref[...]
    ) * SCALE
    z = jax.lax.dot_general(zone_ref[0], wz_ref[...], (((1,), (1,)), ((), ())),
                            preferred_element_type=jnp.float32)
    z = jnp.tanh(z + bz_ref[...])
    x = jax.lax.dot_general(t, z, (((1,), (1,)), ((), ())),
                            preferred_element_type=jnp.float32)
    x = jnp.where(mask_ref[0] != 0, -jnp.inf, x)
    m = jnp.max(x, axis=1, keepdims=True)
    e = jnp.exp(x - m)
    out_ref[...] = (e / jnp.sum(e, axis=1, keepdims=True))[None]


def kernel(txt_embeds, zone_embeds, W_txt, b_txt, W_zone, b_zone, mask):
    out = pl.pallas_call(
        _fused_kernel,
        grid=(B,),
        in_specs=[
            pl.BlockSpec((B, D), lambda b: (0, 0)),
            pl.BlockSpec((1, Z, D), lambda b: (b, 0, 0)),
            pl.BlockSpec((K, D), lambda b: (0, 0)),
            pl.BlockSpec((1, K), lambda b: (0, 0)),
            pl.BlockSpec((K, D), lambda b: (0, 0)),
            pl.BlockSpec((1, K), lambda b: (0, 0)),
            pl.BlockSpec((1, 1, Z), lambda b: (b, 0, 0)),
        ],
        out_specs=pl.BlockSpec((1, 1, Z), lambda b: (b, 0, 0)),
        out_shape=jax.ShapeDtypeStruct((B, 1, Z), jnp.float32),
        compiler_params=pltpu.CompilerParams(
            dimension_semantics=("parallel",)),
    )(txt_embeds, zone_embeds, W_txt, b_txt.reshape(1, K),
      W_zone, b_zone.reshape(1, K), mask.astype(jnp.int32).reshape(B, 1, Z))
    return out.reshape(B, Z)
